# Initial kernel scaffold; baseline (speedup 1.0000x reference)
import jax
import jax.numpy as jnp
from jax.experimental import pallas as pl


def kernel(from_down, from_up, wt, bt, w1, b1, w2, b2):
    raise NotImplementedError("write your pallas kernel here")


# speedup vs baseline: 1.7169x; 1.7169x over previous
import functools

import jax
import jax.numpy as jnp
from jax.experimental import pallas as pl
from jax.experimental.pallas import tpu as pltpu


def _up_pair_mats(wt, Wu):
    Cin, Cout = wt.shape[0], wt.shape[1]
    Wd = 2 * Wu
    wtf = wt.astype(jnp.float32)
    x = jnp.arange(Wu)
    mu = jnp.zeros((2, Wu, Cin, Wd, Cout), jnp.float32)
    for di in range(2):
        for dj in range(2):
            sel = jnp.zeros((Wu, Wd), jnp.float32).at[x, 2 * x + dj].set(1.0)
            mu = mu.at[di].add(
                sel[:, None, :, None] * wtf[:, :, di, dj][None, :, None, :])
    return mu.reshape(2, Wu * Cin, Wd * Cout)


def _band_mats(w_oihw, Wd):
    Cout, Cin = w_oihw.shape[0], w_oihw.shape[1]
    w = jnp.transpose(w_oihw, (2, 3, 1, 0)).astype(jnp.float32)
    mats = []
    for dy in range(3):
        m = jnp.zeros((Wd, Cin, Wd, Cout), jnp.float32)
        for dx in range(3):
            sel = jnp.eye(Wd, Wd, k=-(dx - 1), dtype=jnp.float32)
            m = m + sel[:, None, :, None] * w[dy, dx][None, :, None, :]
        mats.append(m.reshape(Wd * Cin, Wd * Cout))
    return jnp.stack(mats, axis=0)


def _dot(a, b):
    return jnp.dot(a, b, preferred_element_type=jnp.float32)


def _shift_down(x):
    return jnp.concatenate([jnp.zeros_like(x[:1]), x[:-1]], axis=0)


def _shift_up(x):
    return jnp.concatenate([x[1:], jnp.zeros_like(x[:1])], axis=0)


def _fused_kernel(fu_ref, fd_ref, mu_ref, w1u_ref, w1f_ref, w2_ref,
                  btr_ref, b1r_ref, b2r_ref, o_ref):
    bf16 = jnp.bfloat16
    fu = fu_ref[0]
    fd_e = fd_ref[0, :, 0, :]
    fd_o = fd_ref[0, :, 1, :]

    up_e = (_dot(fu, mu_ref[0]) + btr_ref[...]).astype(bf16)
    up_o = (_dot(fu, mu_ref[1]) + btr_ref[...]).astype(bf16)

    uo_m1, fo_m1 = _shift_down(up_o), _shift_down(fd_o)
    ue_p1, fe_p1 = _shift_up(up_e), _shift_up(fd_e)
    h1e = (_dot(uo_m1, w1u_ref[0]) + _dot(fo_m1, w1f_ref[0])
           + _dot(up_e, w1u_ref[1]) + _dot(fd_e, w1f_ref[1])
           + _dot(up_o, w1u_ref[2]) + _dot(fd_o, w1f_ref[2]))
    h1o = (_dot(up_e, w1u_ref[0]) + _dot(fd_e, w1f_ref[0])
           + _dot(up_o, w1u_ref[1]) + _dot(fd_o, w1f_ref[1])
           + _dot(ue_p1, w1u_ref[2]) + _dot(fe_p1, w1f_ref[2]))
    h1e = jnp.maximum(h1e + b1r_ref[...], 0.0).astype(bf16)
    h1o = jnp.maximum(h1o + b1r_ref[...], 0.0).astype(bf16)

    ho_m1 = _shift_down(h1o)
    he_p1 = _shift_up(h1e)
    oe = (_dot(ho_m1, w2_ref[0]) + _dot(h1e, w2_ref[1])
          + _dot(h1o, w2_ref[2]))
    oo = (_dot(h1e, w2_ref[0]) + _dot(h1o, w2_ref[1])
          + _dot(he_p1, w2_ref[2]))
    o_ref[0, :, 0, :] = jnp.maximum(oe + b2r_ref[...], 0.0)
    o_ref[0, :, 1, :] = jnp.maximum(oo + b2r_ref[...], 0.0)


def kernel(from_down, from_up, wt, bt, w1, b1, w2, b2):
    N, Cout, Hd, Wd = from_down.shape
    _, Cin, Hu, Wu = from_up.shape
    bf16 = jnp.bfloat16
    Ku = Wu * Cin
    Nw = Wd * Cout

    fu = jnp.transpose(from_up, (0, 2, 3, 1)).reshape(N, Hu, Ku).astype(bf16)
    fd = jnp.transpose(from_down, (0, 2, 3, 1)).reshape(N, Hu, 2, Nw).astype(bf16)

    mu = _up_pair_mats(wt, Wu).astype(bf16)
    w1u = _band_mats(w1[:, :Cout], Wd).astype(bf16)
    w1f = _band_mats(w1[:, Cout:], Wd).astype(bf16)
    w2b = _band_mats(w2, Wd).astype(bf16)
    btr = jnp.tile(bt.astype(jnp.float32), Wd).reshape(1, Nw)
    b1r = jnp.tile(b1.astype(jnp.float32), Wd).reshape(1, Nw)
    b2r = jnp.tile(b2.astype(jnp.float32), Wd).reshape(1, Nw)

    out = pl.pallas_call(
        _fused_kernel,
        out_shape=jax.ShapeDtypeStruct((N, Hu, 2, Nw), jnp.float32),
        grid=(N,),
        in_specs=[
            pl.BlockSpec((1, Hu, Ku), lambda n: (n, 0, 0)),
            pl.BlockSpec((1, Hu, 2, Nw), lambda n: (n, 0, 0, 0)),
            pl.BlockSpec((2, Ku, Nw), lambda n: (0, 0, 0)),
            pl.BlockSpec((3, Nw, Nw), lambda n: (0, 0, 0)),
            pl.BlockSpec((3, Nw, Nw), lambda n: (0, 0, 0)),
            pl.BlockSpec((3, Nw, Nw), lambda n: (0, 0, 0)),
            pl.BlockSpec((1, Nw), lambda n: (0, 0)),
            pl.BlockSpec((1, Nw), lambda n: (0, 0)),
            pl.BlockSpec((1, Nw), lambda n: (0, 0)),
        ],
        out_specs=pl.BlockSpec((1, Hu, 2, Nw), lambda n: (n, 0, 0, 0)),
        compiler_params=pltpu.CompilerParams(
            dimension_semantics=("parallel",),
            vmem_limit_bytes=64 * 1024 * 1024,
        ),
    )(fu, fd, mu, w1u, w1f, w2b, btr, b1r, b2r)

    out = out.reshape(N, Hd, Wd, Cout)
    return jnp.transpose(out, (0, 3, 1, 2))
